# speedup vs baseline: 685.1829x; 685.1829x over previous
import jax
import jax.numpy as jnp
from jax.experimental import pallas as pl
from jax.experimental.pallas import tpu as pltpu

GRID_SIZE = 2
AREA_SPAN = 1.6

N = 4096
H = 256
BI = 512
BJ = 512
NIB = N // BI
NJB = N // BJ
KMAX = NJB


def _body(jblk_ref, nk_ref, ht_ref, pxc_ref, pyc_ref, ssc_ref,
          pxr_ref, pyr_ref, ssr_ref, out_ref):
    ib = pl.program_id(0)
    k = pl.program_id(1)

    @pl.when(k == 0)
    def _init():
        out_ref[...] = jnp.zeros_like(out_ref)

    @pl.when(k < nk_ref[ib])
    def _compute():
        jb = jblk_ref[ib, k]
        half = AREA_SPAN / 2.0
        cell = AREA_SPAN / GRID_SIZE
        eps = 0.01

        pxi = pxc_ref[...]
        pyi = pyc_ref[...]
        ssi = ssc_ref[...]
        pxj = pxr_ref[0]
        pyj = pyr_ref[0]
        ssj = ssr_ref[0]

        relx = pxj - pxi
        rely = pyj - pyi
        box = ((relx < half - eps) & (relx > -(half - eps))
               & (rely < half - eps) & (rely > -(half - eps)))
        same = ssj == ssi
        iglob = ib * BI + jax.lax.broadcasted_iota(jnp.int32, (BI, BJ), 0)
        jglob = jb * BJ + jax.lax.broadcasted_iota(jnp.int32, (BI, BJ), 1)
        within = box & same & (iglob != jglob)

        gx = jnp.floor((relx + half) / cell)
        gy = jnp.floor((rely + half) / cell)
        gid = gx * GRID_SIZE + gy

        htb = ht_ref[...]
        for b in range(GRID_SIZE * GRID_SIZE):
            mb = jnp.where(within & (gid == float(b)), 1.0, 0.0)
            acc = jnp.dot(mb, htb, preferred_element_type=jnp.float32)
            out_ref[:, b, :] += acc


def kernel(ht, pos_t, same_scene_mask):
    ht2 = ht.reshape(N, H).astype(jnp.float32)
    pos = pos_t.reshape(N, 2).astype(jnp.float32)
    ssm = same_scene_mask.reshape(N).astype(jnp.int32)

    px = pos[:, 0]
    py = pos[:, 1]
    pxc = px.reshape(N, 1)
    pyc = py.reshape(N, 1)
    ssc = ssm.reshape(N, 1)
    pxr = px.reshape(NJB, 1, BJ)
    pyr = py.reshape(NJB, 1, BJ)
    ssr = ssm.reshape(NJB, 1, BJ)

    resh = ssm.reshape(NIB, BI)
    first = resh[:, 0]
    last = resh[:, -1]
    jstart = jnp.searchsorted(ssm, first, side="left")
    jend = jnp.searchsorted(ssm, last, side="right")
    jb0 = (jstart // BJ).astype(jnp.int32)
    jb1 = ((jend - 1) // BJ).astype(jnp.int32)
    nk = jb1 - jb0 + 1
    ks = jnp.arange(KMAX, dtype=jnp.int32)
    jblk = jnp.minimum(jb0[:, None] + ks[None, :], jb1[:, None]).astype(jnp.int32)

    grid_spec = pltpu.PrefetchScalarGridSpec(
        num_scalar_prefetch=2,
        grid=(NIB, KMAX),
        in_specs=[
            pl.BlockSpec((BJ, H), lambda ib, k, jblk, nk: (jblk[ib, k], 0)),
            pl.BlockSpec((BI, 1), lambda ib, k, jblk, nk: (ib, 0)),
            pl.BlockSpec((BI, 1), lambda ib, k, jblk, nk: (ib, 0)),
            pl.BlockSpec((BI, 1), lambda ib, k, jblk, nk: (ib, 0)),
            pl.BlockSpec((1, 1, BJ), lambda ib, k, jblk, nk: (jblk[ib, k], 0, 0)),
            pl.BlockSpec((1, 1, BJ), lambda ib, k, jblk, nk: (jblk[ib, k], 0, 0)),
            pl.BlockSpec((1, 1, BJ), lambda ib, k, jblk, nk: (jblk[ib, k], 0, 0)),
        ],
        out_specs=pl.BlockSpec((BI, GRID_SIZE * GRID_SIZE, H),
                               lambda ib, k, jblk, nk: (ib, 0, 0)),
    )

    out = pl.pallas_call(
        _body,
        grid_spec=grid_spec,
        out_shape=jax.ShapeDtypeStruct((N, GRID_SIZE * GRID_SIZE, H),
                                       jnp.float32),
    )(jblk, nk, ht2, pxc, pyc, ssc, pxr, pyr, ssr)
    return out
